# baseline (speedup 1.0000x reference)
import functools

import jax
import jax.numpy as jnp
from jax import lax
from jax.experimental import pallas as pl
from jax.experimental.pallas import tpu as pltpu
from jax.experimental.pallas import tpu_sc as plsc

B = 128
D = 128
S = 2048
E = 8
KW = 8
BB = 64
NB = B // BB
SBS = 128
NBLK = S // SBS

NC = 2
NS = 16
NW = NC * NS
RPW = (B * KW) // NW
LANES = 16


def _gelu_exact(x):
    return x * 0.5 * (1.0 + lax.erf(x / (2.0 ** 0.5)))


def _fused_body(s_ref, wq_ref, mk_ref, mv_ref, wr_ref, br_ref, w1_ref, w2_ref,
                wg_ref, bg_ref, wwl_ref, bwl_ref, wk_ref, wv_ref,
                mkout_ref, mvout_ref, snew_ref, a3_ref, ck3_ref, cv3_ref,
                fidx_ref, att_sc, q_sc, racc_sc, wread_sc):
    bi = pl.program_id(0)
    j = pl.program_id(1)
    brow = pl.ds(bi * BB, BB)

    @pl.when((bi == 0) & (j == 0))
    def _():
        q_sc[...] = lax.dot_general(
            s_ref[...], wq_ref[...], (((1,), (1,)), ((), ())))
        racc_sc[...] = jnp.zeros_like(racc_sc)

    @pl.when(j < NBLK)
    def _():
        mk = mk_ref[...]
        mkout_ref[...] = mk
        att = jnp.sum(mk * q_sc[brow, :][:, None, :], axis=-1) / (D ** 0.5)
        att_sc[brow, pl.ds(j * SBS, SBS)] = att

    @pl.when(j == NBLK)
    def _():
        a = att_sc[brow, :]
        m = jnp.max(a, axis=-1, keepdims=True)
        e = jnp.exp(a - m)
        wread_sc[brow, :] = e / jnp.sum(e, axis=-1, keepdims=True)

    @pl.when(j >= NBLK)
    def _():
        mv = mv_ref[...]
        mvout_ref[...] = mv
        w = wread_sc[brow, pl.ds((j - NBLK) * SBS, SBS)]
        racc_sc[brow, :] += jnp.sum(mv * w[:, :, None], axis=1)

    @pl.when((bi == NB - 1) & (j == 2 * NBLK - 1))
    def _():
        h = s_ref[...] + racc_sc[...]
        logits = lax.dot_general(
            h, wr_ref[...], (((1,), (1,)), ((), ()))) + br_ref[...]
        lm = jnp.max(logits, axis=-1, keepdims=True)
        le = jnp.exp(logits - lm)
        p = le / jnp.sum(le, axis=-1, keepdims=True)

        ei = lax.broadcasted_iota(jnp.int32, (B, E), 1)
        work = p
        wsel = jnp.zeros((B, E), jnp.float32)
        for _ in range(2):
            m = jnp.max(work, axis=-1, keepdims=True)
            ix = jnp.min(jnp.where(work == m, ei, E), axis=-1, keepdims=True)
            sel = ei == ix
            wsel = wsel + jnp.where(sel, m, 0.0)
            work = jnp.where(sel, -jnp.inf, work)
        wsel = wsel / (jnp.sum(wsel, axis=-1, keepdims=True) + 1e-8)

        y = jnp.zeros((B, D), jnp.float32)
        for e in range(E):
            hid = lax.dot_general(
                h, w1_ref[e], (((1,), (1,)), ((), ())))
            hid = _gelu_exact(hid)
            eo = lax.dot_general(
                hid, w2_ref[e], (((1,), (1,)), ((), ())))
            y = y + wsel[:, e][:, None] * eo
        s_new = h + y
        snew_ref[...] = s_new

        gl = jnp.sum(s_new * wg_ref[...], axis=-1, keepdims=True) + bg_ref[0, 0]
        gate = 1.0 / (1.0 + jnp.exp(-gl))

        wl = lax.dot_general(
            s_new, wwl_ref[...], (((1,), (1,)), ((), ()))) + bwl_ref[...]
        si = lax.broadcasted_iota(jnp.int32, (B, S), 1)
        work = wl
        vals, idxs = [], []
        for _ in range(KW):
            m = jnp.max(work, axis=-1, keepdims=True)
            ix = jnp.min(jnp.where(work == m, si, S), axis=-1, keepdims=True)
            vals.append(m)
            idxs.append(ix)
            work = jnp.where(si == ix, -jnp.inf, work)
        vv = jnp.concatenate(vals, axis=1)
        ii = jnp.concatenate(idxs, axis=1)
        mm = jnp.max(vv, axis=-1, keepdims=True)
        ee = jnp.exp(vv - mm)
        ws = ee / jnp.sum(ee, axis=-1, keepdims=True)
        gw = gate * ws

        a3_ref[...] = jnp.broadcast_to((1.0 - gw)[:, :, None], (B, KW, LANES))
        wk = lax.dot_general(s_new, wk_ref[...], (((1,), (1,)), ((), ())))
        wv = lax.dot_general(s_new, wv_ref[...], (((1,), (1,)), ((), ())))
        ck3_ref[...] = gw[:, :, None] * wk[:, None, :]
        cv3_ref[...] = gw[:, :, None] * wv[:, None, :]
        fidx_ref[...] = ii + lax.broadcasted_iota(jnp.int32, (B, KW), 0) * S


def _fused(s, mem_k, mem_v, Wq, W_wl, b_wl, Wk, Wv, Wr, br, W1, W2, Wg, bg):
    const = lambda shape: pl.BlockSpec(shape, lambda bi, j: tuple(
        0 for _ in shape))
    return pl.pallas_call(
        _fused_body,
        grid=(NB, 2 * NBLK),
        in_specs=[
            const((B, D)),
            const((D, D)),
            pl.BlockSpec((BB, SBS, D),
                         lambda bi, j: (bi, jnp.minimum(j, NBLK - 1), 0)),
            pl.BlockSpec((BB, SBS, D),
                         lambda bi, j: (bi, jnp.maximum(j - NBLK, 0), 0)),
            const((E, D)),
            const((1, E)),
            const((E, 4 * D, D)),
            const((E, D, 4 * D)),
            const((1, D)),
            const((1, 1)),
            const((S, D)),
            const((1, S)),
            const((D, D)),
            const((D, D)),
        ],
        out_specs=[
            pl.BlockSpec((BB, SBS, D),
                         lambda bi, j: (bi, jnp.minimum(j, NBLK - 1), 0)),
            pl.BlockSpec((BB, SBS, D),
                         lambda bi, j: (bi, jnp.maximum(j - NBLK, 0), 0)),
            const((B, D)),
            const((B, KW, LANES)),
            const((B, KW, D)),
            const((B, KW, D)),
            const((B, KW)),
        ],
        out_shape=[
            jax.ShapeDtypeStruct((B, S, D), jnp.float32),
            jax.ShapeDtypeStruct((B, S, D), jnp.float32),
            jax.ShapeDtypeStruct((B, D), jnp.float32),
            jax.ShapeDtypeStruct((B, KW, LANES), jnp.float32),
            jax.ShapeDtypeStruct((B, KW, D), jnp.float32),
            jax.ShapeDtypeStruct((B, KW, D), jnp.float32),
            jax.ShapeDtypeStruct((B, KW), jnp.int32),
        ],
        scratch_shapes=[
            pltpu.VMEM((B, S), jnp.float32),
            pltpu.VMEM((B, D), jnp.float32),
            pltpu.VMEM((B, D), jnp.float32),
            pltpu.VMEM((B, S), jnp.float32),
        ],
        compiler_params=pltpu.CompilerParams(
            dimension_semantics=("arbitrary", "arbitrary")),
    )(s, Wq, mem_k, mem_v, Wr, br.reshape(1, E), W1, W2, Wg,
      bg.reshape(1, 1), W_wl, b_wl.reshape(1, S), Wk, Wv)


def _sc_scatter_body(memk_ref, memv_ref, fidx_hbm, a_hbm, ck_hbm, cv_hbm,
                     idx_v, a_v, ck_v, cv_v, oldk_v, oldv_v,
                     sem1, sem2, sem3):
    wid = lax.axis_index("s") * NC + lax.axis_index("c")
    base = wid * RPW
    cpi = pltpu.async_copy(fidx_hbm.at[pl.ds(base, RPW)], idx_v, sem3)
    cpa = pltpu.async_copy(a_hbm.at[pl.ds(base, RPW)], a_v, sem1)
    cpk = pltpu.async_copy(ck_hbm.at[pl.ds(base, RPW)], ck_v, sem2)
    cpv = pltpu.async_copy(cv_hbm.at[pl.ds(base, RPW)], cv_v, sem2)
    cpi.wait()
    cp1 = pltpu.async_copy(memk_ref.at[idx_v], oldk_v, sem3)
    cp2 = pltpu.async_copy(memv_ref.at[idx_v], oldv_v, sem3)
    cpa.wait()
    cpk.wait()
    cpv.wait()
    cp1.wait()
    cp2.wait()

    def row(j, carry):
        av = a_v[j, :]
        for hh in range(D // LANES):
            sl = pl.ds(hh * LANES, LANES)
            oldk_v[j, sl] = av * oldk_v[j, sl] + ck_v[j, sl]
            oldv_v[j, sl] = av * oldv_v[j, sl] + cv_v[j, sl]
        return carry

    lax.fori_loop(0, RPW, row, 0)

    cp3 = pltpu.async_copy(oldk_v, memk_ref.at[idx_v], sem1)
    cp4 = pltpu.async_copy(oldv_v, memv_ref.at[idx_v], sem2)
    cp3.wait()
    cp4.wait()


@functools.cache
def _make_sc_scatter():
    return pl.kernel(
        _sc_scatter_body,
        mesh=plsc.VectorSubcoreMesh(core_axis_name="c", subcore_axis_name="s"),
        scratch_types=[
            pltpu.VMEM((RPW,), jnp.int32),
            pltpu.VMEM((RPW, LANES), jnp.float32),
            pltpu.VMEM((RPW, D), jnp.float32),
            pltpu.VMEM((RPW, D), jnp.float32),
            pltpu.VMEM((RPW, D), jnp.float32),
            pltpu.VMEM((RPW, D), jnp.float32),
            pltpu.SemaphoreType.DMA,
            pltpu.SemaphoreType.DMA,
            pltpu.SemaphoreType.DMA,
        ],
    )


def _sc_scatter(mkf, mvf, fidx, a2, ck2, cv2):
    _make_sc_scatter()(mkf, mvf, fidx, a2, ck2, cv2)


def kernel(s, mem_k, mem_v, Wq, W_wl, b_wl, Wk, Wv, Wr, br, W1, W2, Wg, bg):
    (mk_copy, mv_copy, s_new, A3, Ck3, Cv3, fidx) = _fused(
        s, mem_k, mem_v, Wq, W_wl, b_wl, Wk, Wv, Wr, br, W1, W2, Wg, bg)

    mkf = jax.new_ref(mk_copy.reshape(B * S, D))
    mvf = jax.new_ref(mv_copy.reshape(B * S, D))
    _sc_scatter(mkf, mvf,
                fidx.reshape(B * KW),
                A3.reshape(B * KW, LANES),
                Ck3.reshape(B * KW, D),
                Cv3.reshape(B * KW, D))
    mem_k_new = jax.freeze(mkf).reshape(B, S, D)
    mem_v_new = jax.freeze(mvf).reshape(B, S, D)
    return s_new, mem_k_new, mem_v_new


# speedup vs baseline: 1.0202x; 1.0202x over previous
import functools

import jax
import jax.numpy as jnp
from jax import lax
from jax.experimental import pallas as pl
from jax.experimental.pallas import tpu as pltpu
from jax.experimental.pallas import tpu_sc as plsc

B = 128
D = 128
S = 2048
E = 8
KW = 8
BS = 128
NBLK = S // BS

NC = 2
NS = 16
NW = NC * NS
RPW = (B * KW) // NW
LANES = 16


def _gelu_exact(x):
    return x * 0.5 * (1.0 + lax.erf(x / (2.0 ** 0.5)))


def _sweep1_body(s_ref, wq_ref, mk_ref, mkout_ref, wread_ref, att_sc, q_sc):
    j = pl.program_id(0)

    @pl.when(j == 0)
    def _():
        q_sc[...] = lax.dot_general(
            s_ref[...], wq_ref[...], (((1,), (1,)), ((), ())))

    mk = mk_ref[...]
    mkout_ref[...] = mk
    att = jnp.sum(mk * q_sc[...][:, None, :], axis=-1) / (D ** 0.5)
    att_sc[:, pl.ds(j * BS, BS)] = att

    @pl.when(j == NBLK - 1)
    def _():
        a = att_sc[...]
        m = jnp.max(a, axis=-1, keepdims=True)
        e = jnp.exp(a - m)
        wread_ref[...] = e / jnp.sum(e, axis=-1, keepdims=True)


def _sweep1(s, Wq, mem_k):
    return pl.pallas_call(
        _sweep1_body,
        grid=(NBLK,),
        in_specs=[
            pl.BlockSpec((B, D), lambda j: (0, 0)),
            pl.BlockSpec((D, D), lambda j: (0, 0)),
            pl.BlockSpec((B, BS, D), lambda j: (0, j, 0)),
        ],
        out_specs=[
            pl.BlockSpec((B, BS, D), lambda j: (0, j, 0)),
            pl.BlockSpec((B, S), lambda j: (0, 0)),
        ],
        out_shape=[
            jax.ShapeDtypeStruct((B, S, D), jnp.float32),
            jax.ShapeDtypeStruct((B, S), jnp.float32),
        ],
        scratch_shapes=[
            pltpu.VMEM((B, S), jnp.float32),
            pltpu.VMEM((B, D), jnp.float32),
        ],
        compiler_params=pltpu.CompilerParams(
            dimension_semantics=("arbitrary",)),
    )(s, Wq, mem_k)


def _sweep2_body(wread_ref, s_ref, mv_ref, wr_ref, br_ref, w1_ref, w2_ref,
                 wg_ref, bg_ref, wwl_ref, bwl_ref, wk_ref, wv_ref,
                 mvout_ref, snew_ref, a3_ref, ck3_ref, cv3_ref, fidx_ref,
                 racc_sc):
    j = pl.program_id(0)

    @pl.when(j == 0)
    def _():
        racc_sc[...] = jnp.zeros_like(racc_sc)

    mv = mv_ref[...]
    mvout_ref[...] = mv
    w = wread_ref[:, pl.ds(j * BS, BS)]
    racc_sc[...] += jnp.sum(mv * w[:, :, None], axis=1)

    @pl.when(j == NBLK - 1)
    def _():
        h = s_ref[...] + racc_sc[...]
        logits = lax.dot_general(
            h, wr_ref[...], (((1,), (1,)), ((), ()))) + br_ref[...]
        lm = jnp.max(logits, axis=-1, keepdims=True)
        le = jnp.exp(logits - lm)
        p = le / jnp.sum(le, axis=-1, keepdims=True)

        ei = lax.broadcasted_iota(jnp.int32, (B, E), 1)
        work = p
        wsel = jnp.zeros((B, E), jnp.float32)
        for _ in range(2):
            m = jnp.max(work, axis=-1, keepdims=True)
            ix = jnp.min(jnp.where(work == m, ei, E), axis=-1, keepdims=True)
            sel = ei == ix
            wsel = wsel + jnp.where(sel, m, 0.0)
            work = jnp.where(sel, -jnp.inf, work)
        wsel = wsel / (jnp.sum(wsel, axis=-1, keepdims=True) + 1e-8)

        y = jnp.zeros((B, D), jnp.float32)
        for e in range(E):
            hid = lax.dot_general(
                h, w1_ref[e], (((1,), (1,)), ((), ())))
            hid = _gelu_exact(hid)
            eo = lax.dot_general(
                hid, w2_ref[e], (((1,), (1,)), ((), ())))
            y = y + wsel[:, e][:, None] * eo
        s_new = h + y
        snew_ref[...] = s_new

        gl = jnp.sum(s_new * wg_ref[...], axis=-1,
                     keepdims=True) + bg_ref[0, 0]
        gate = 1.0 / (1.0 + jnp.exp(-gl))

        wl = lax.dot_general(
            s_new, wwl_ref[...], (((1,), (1,)), ((), ()))) + bwl_ref[...]
        si = lax.broadcasted_iota(jnp.int32, (B, S), 1)
        work = wl
        vals, idxs = [], []
        for _ in range(KW):
            m = jnp.max(work, axis=-1, keepdims=True)
            ix = jnp.min(jnp.where(work == m, si, S), axis=-1, keepdims=True)
            vals.append(m)
            idxs.append(ix)
            work = jnp.where(si == ix, -jnp.inf, work)
        vv = jnp.concatenate(vals, axis=1)
        ii = jnp.concatenate(idxs, axis=1)
        mm = jnp.max(vv, axis=-1, keepdims=True)
        ee = jnp.exp(vv - mm)
        ws = ee / jnp.sum(ee, axis=-1, keepdims=True)
        gw = gate * ws

        a3_ref[...] = jnp.broadcast_to(
            (1.0 - gw)[:, :, None], (B, KW, LANES))
        wk = lax.dot_general(s_new, wk_ref[...], (((1,), (1,)), ((), ())))
        wv = lax.dot_general(s_new, wv_ref[...], (((1,), (1,)), ((), ())))
        ck3_ref[...] = gw[:, :, None] * wk[:, None, :]
        cv3_ref[...] = gw[:, :, None] * wv[:, None, :]
        fidx_ref[...] = ii + lax.broadcasted_iota(jnp.int32, (B, KW), 0) * S


def _sweep2(w_read, s, mem_v, Wr, br, W1, W2, Wg, bg, W_wl, b_wl, Wk, Wv):
    const = lambda shape: pl.BlockSpec(shape, lambda j: tuple(
        0 for _ in shape))
    return pl.pallas_call(
        _sweep2_body,
        grid=(NBLK,),
        in_specs=[
            const((B, S)),
            const((B, D)),
            pl.BlockSpec((B, BS, D), lambda j: (0, j, 0)),
            const((E, D)),
            const((1, E)),
            const((E, 4 * D, D)),
            const((E, D, 4 * D)),
            const((1, D)),
            const((1, 1)),
            const((S, D)),
            const((1, S)),
            const((D, D)),
            const((D, D)),
        ],
        out_specs=[
            pl.BlockSpec((B, BS, D), lambda j: (0, j, 0)),
            const((B, D)),
            const((B, KW, LANES)),
            const((B, KW, D)),
            const((B, KW, D)),
            const((B, KW)),
        ],
        out_shape=[
            jax.ShapeDtypeStruct((B, S, D), jnp.float32),
            jax.ShapeDtypeStruct((B, D), jnp.float32),
            jax.ShapeDtypeStruct((B, KW, LANES), jnp.float32),
            jax.ShapeDtypeStruct((B, KW, D), jnp.float32),
            jax.ShapeDtypeStruct((B, KW, D), jnp.float32),
            jax.ShapeDtypeStruct((B, KW), jnp.int32),
        ],
        scratch_shapes=[pltpu.VMEM((B, D), jnp.float32)],
        compiler_params=pltpu.CompilerParams(
            dimension_semantics=("arbitrary",)),
    )(w_read, s, mem_v, Wr, br.reshape(1, E), W1, W2, Wg,
      bg.reshape(1, 1), W_wl, b_wl.reshape(1, S), Wk, Wv)


def _sc_scatter_body(memk_ref, memv_ref, fidx_hbm, a_hbm, ck_hbm, cv_hbm,
                     idx_v, a_v, ck_v, cv_v, oldk_v, oldv_v,
                     sem1, sem2, sem3):
    wid = lax.axis_index("s") * NC + lax.axis_index("c")
    base = wid * RPW
    cpi = pltpu.async_copy(fidx_hbm.at[pl.ds(base, RPW)], idx_v, sem3)
    cpa = pltpu.async_copy(a_hbm.at[pl.ds(base, RPW)], a_v, sem1)
    cpk = pltpu.async_copy(ck_hbm.at[pl.ds(base, RPW)], ck_v, sem2)
    cpv = pltpu.async_copy(cv_hbm.at[pl.ds(base, RPW)], cv_v, sem2)
    cpi.wait()
    cp1 = pltpu.async_copy(memk_ref.at[idx_v], oldk_v, sem3)
    cp2 = pltpu.async_copy(memv_ref.at[idx_v], oldv_v, sem3)
    cpa.wait()
    cpk.wait()
    cpv.wait()
    cp1.wait()
    cp2.wait()

    def row(j, carry):
        av = a_v[j, :]
        for hh in range(D // LANES):
            sl = pl.ds(hh * LANES, LANES)
            oldk_v[j, sl] = av * oldk_v[j, sl] + ck_v[j, sl]
            oldv_v[j, sl] = av * oldv_v[j, sl] + cv_v[j, sl]
        return carry

    lax.fori_loop(0, RPW, row, 0)

    cp3 = pltpu.async_copy(oldk_v, memk_ref.at[idx_v], sem1)
    cp4 = pltpu.async_copy(oldv_v, memv_ref.at[idx_v], sem2)
    cp3.wait()
    cp4.wait()


@functools.cache
def _make_sc_scatter():
    return pl.kernel(
        _sc_scatter_body,
        mesh=plsc.VectorSubcoreMesh(core_axis_name="c", subcore_axis_name="s"),
        scratch_types=[
            pltpu.VMEM((RPW,), jnp.int32),
            pltpu.VMEM((RPW, LANES), jnp.float32),
            pltpu.VMEM((RPW, D), jnp.float32),
            pltpu.VMEM((RPW, D), jnp.float32),
            pltpu.VMEM((RPW, D), jnp.float32),
            pltpu.VMEM((RPW, D), jnp.float32),
            pltpu.SemaphoreType.DMA,
            pltpu.SemaphoreType.DMA,
            pltpu.SemaphoreType.DMA,
        ],
    )


def _sc_scatter(mkf, mvf, fidx, a2, ck2, cv2):
    _make_sc_scatter()(mkf, mvf, fidx, a2, ck2, cv2)


def kernel(s, mem_k, mem_v, Wq, W_wl, b_wl, Wk, Wv, Wr, br, W1, W2, Wg, bg):
    mk_copy, w_read = _sweep1(s, Wq, mem_k)
    (mv_copy, s_new, A3, Ck3, Cv3, fidx) = _sweep2(
        w_read, s, mem_v, Wr, br, W1, W2, Wg, bg, W_wl, b_wl, Wk, Wv)

    mkf = jax.new_ref(mk_copy.reshape(B * S, D))
    mvf = jax.new_ref(mv_copy.reshape(B * S, D))
    _sc_scatter(mkf, mvf,
                fidx.reshape(B * KW),
                A3.reshape(B * KW, LANES),
                Ck3.reshape(B * KW, D),
                Cv3.reshape(B * KW, D))
    mem_k_new = jax.freeze(mkf).reshape(B, S, D)
    mem_v_new = jax.freeze(mvf).reshape(B, S, D)
    return s_new, mem_k_new, mem_v_new


# speedup vs baseline: 1.0407x; 1.0201x over previous
import functools

import jax
import jax.numpy as jnp
from jax import lax
from jax.experimental import pallas as pl
from jax.experimental.pallas import tpu as pltpu
from jax.experimental.pallas import tpu_sc as plsc

B = 128
D = 128
S = 2048
E = 8
KW = 8
BS = 64
NBLK = S // BS

NC = 2
NS = 16
NW = NC * NS
RPW = (B * KW) // NW
LANES = 16


def _gelu_exact(x):
    return x * 0.5 * (1.0 + lax.erf(x / (2.0 ** 0.5)))


def _stream_body(s_ref, wq_ref, mk_ref, mv_ref, wr_ref, br_ref, w1_ref,
                 w2_ref, wg_ref, bg_ref, wwl_ref, bwl_ref, wk_ref, wv_ref,
                 mkout_ref, mvout_ref, snew_ref, a3_ref, ck3_ref, cv3_ref,
                 fidx_ref, q_sc, m_sc, l_sc, racc_sc):
    j = pl.program_id(0)

    @pl.when(j == 0)
    def _():
        q_sc[...] = lax.dot_general(
            s_ref[...], wq_ref[...], (((1,), (1,)), ((), ())))
        m_sc[...] = jnp.full_like(m_sc, -1e30)
        l_sc[...] = jnp.zeros_like(l_sc)
        racc_sc[...] = jnp.zeros_like(racc_sc)

    mk = mk_ref[...]
    mkout_ref[...] = mk
    att = jnp.sum(mk * q_sc[...][:, None, :], axis=-1) / (D ** 0.5)

    mloc = jnp.max(att, axis=-1, keepdims=True)
    mnew = jnp.maximum(m_sc[...], mloc)
    alpha = jnp.exp(m_sc[...] - mnew)
    ex = jnp.exp(att - mnew)
    l_sc[...] = l_sc[...] * alpha + jnp.sum(ex, axis=-1, keepdims=True)
    m_sc[...] = mnew

    mv = mv_ref[...]
    mvout_ref[...] = mv
    racc_sc[...] = racc_sc[...] * alpha + jnp.sum(mv * ex[:, :, None], axis=1)

    @pl.when(j == NBLK - 1)
    def _():
        r = racc_sc[...] / l_sc[...]
        h = s_ref[...] + r
        logits = lax.dot_general(
            h, wr_ref[...], (((1,), (1,)), ((), ()))) + br_ref[...]
        lm = jnp.max(logits, axis=-1, keepdims=True)
        le = jnp.exp(logits - lm)
        p = le / jnp.sum(le, axis=-1, keepdims=True)

        ei = lax.broadcasted_iota(jnp.int32, (B, E), 1)
        work = p
        wsel = jnp.zeros((B, E), jnp.float32)
        for _ in range(2):
            m = jnp.max(work, axis=-1, keepdims=True)
            ix = jnp.min(jnp.where(work == m, ei, E), axis=-1, keepdims=True)
            sel = ei == ix
            wsel = wsel + jnp.where(sel, m, 0.0)
            work = jnp.where(sel, -jnp.inf, work)
        wsel = wsel / (jnp.sum(wsel, axis=-1, keepdims=True) + 1e-8)

        y = jnp.zeros((B, D), jnp.float32)
        for e in range(E):
            hid = lax.dot_general(
                h, w1_ref[e], (((1,), (1,)), ((), ())))
            hid = _gelu_exact(hid)
            eo = lax.dot_general(
                hid, w2_ref[e], (((1,), (1,)), ((), ())))
            y = y + wsel[:, e][:, None] * eo
        s_new = h + y
        snew_ref[...] = s_new

        gl = jnp.sum(s_new * wg_ref[...], axis=-1,
                     keepdims=True) + bg_ref[0, 0]
        gate = 1.0 / (1.0 + jnp.exp(-gl))

        wl = lax.dot_general(
            s_new, wwl_ref[...], (((1,), (1,)), ((), ()))) + bwl_ref[...]
        si = lax.broadcasted_iota(jnp.int32, (B, S), 1)
        work = wl
        vals, idxs = [], []
        for _ in range(KW):
            m = jnp.max(work, axis=-1, keepdims=True)
            ix = jnp.min(jnp.where(work == m, si, S), axis=-1, keepdims=True)
            vals.append(m)
            idxs.append(ix)
            work = jnp.where(si == ix, -jnp.inf, work)
        vv = jnp.concatenate(vals, axis=1)
        ii = jnp.concatenate(idxs, axis=1)
        mm = jnp.max(vv, axis=-1, keepdims=True)
        ee = jnp.exp(vv - mm)
        ws = ee / jnp.sum(ee, axis=-1, keepdims=True)
        gw = gate * ws

        a3_ref[...] = jnp.broadcast_to(
            (1.0 - gw)[:, :, None], (B, KW, LANES))
        wk = lax.dot_general(s_new, wk_ref[...], (((1,), (1,)), ((), ())))
        wv = lax.dot_general(s_new, wv_ref[...], (((1,), (1,)), ((), ())))
        ck3_ref[...] = gw[:, :, None] * wk[:, None, :]
        cv3_ref[...] = gw[:, :, None] * wv[:, None, :]
        fidx_ref[...] = ii + lax.broadcasted_iota(jnp.int32, (B, KW), 0) * S


def _stream(s, mem_k, mem_v, Wq, W_wl, b_wl, Wk, Wv, Wr, br, W1, W2, Wg, bg):
    const = lambda shape: pl.BlockSpec(shape, lambda j: tuple(
        0 for _ in shape))
    return pl.pallas_call(
        _stream_body,
        grid=(NBLK,),
        in_specs=[
            const((B, D)),
            const((D, D)),
            pl.BlockSpec((B, BS, D), lambda j: (0, j, 0)),
            pl.BlockSpec((B, BS, D), lambda j: (0, j, 0)),
            const((E, D)),
            const((1, E)),
            const((E, 4 * D, D)),
            const((E, D, 4 * D)),
            const((1, D)),
            const((1, 1)),
            const((S, D)),
            const((1, S)),
            const((D, D)),
            const((D, D)),
        ],
        out_specs=[
            pl.BlockSpec((B, BS, D), lambda j: (0, j, 0)),
            pl.BlockSpec((B, BS, D), lambda j: (0, j, 0)),
            const((B, D)),
            const((B, KW, LANES)),
            const((B, KW, D)),
            const((B, KW, D)),
            const((B, KW)),
        ],
        out_shape=[
            jax.ShapeDtypeStruct((B, S, D), jnp.float32),
            jax.ShapeDtypeStruct((B, S, D), jnp.float32),
            jax.ShapeDtypeStruct((B, D), jnp.float32),
            jax.ShapeDtypeStruct((B, KW, LANES), jnp.float32),
            jax.ShapeDtypeStruct((B, KW, D), jnp.float32),
            jax.ShapeDtypeStruct((B, KW, D), jnp.float32),
            jax.ShapeDtypeStruct((B, KW), jnp.int32),
        ],
        scratch_shapes=[
            pltpu.VMEM((B, D), jnp.float32),
            pltpu.VMEM((B, 1), jnp.float32),
            pltpu.VMEM((B, 1), jnp.float32),
            pltpu.VMEM((B, D), jnp.float32),
        ],
        compiler_params=pltpu.CompilerParams(
            dimension_semantics=("arbitrary",)),
    )(s, Wq, mem_k, mem_v, Wr, br.reshape(1, E), W1, W2, Wg,
      bg.reshape(1, 1), W_wl, b_wl.reshape(1, S), Wk, Wv)


def _sc_scatter_body(memk_ref, memv_ref, fidx_hbm, a_hbm, ck_hbm, cv_hbm,
                     idx_v, a_v, ck_v, cv_v, oldk_v, oldv_v,
                     sem1, sem2, sem3):
    wid = lax.axis_index("s") * NC + lax.axis_index("c")
    base = wid * RPW
    cpi = pltpu.async_copy(fidx_hbm.at[pl.ds(base, RPW)], idx_v, sem3)
    cpa = pltpu.async_copy(a_hbm.at[pl.ds(base, RPW)], a_v, sem1)
    cpk = pltpu.async_copy(ck_hbm.at[pl.ds(base, RPW)], ck_v, sem2)
    cpv = pltpu.async_copy(cv_hbm.at[pl.ds(base, RPW)], cv_v, sem2)
    cpi.wait()
    cp1 = pltpu.async_copy(memk_ref.at[idx_v], oldk_v, sem3)
    cp2 = pltpu.async_copy(memv_ref.at[idx_v], oldv_v, sem3)
    cpa.wait()
    cpk.wait()
    cpv.wait()
    cp1.wait()
    cp2.wait()

    def row(j, carry):
        av = a_v[j, :]
        for hh in range(D // LANES):
            sl = pl.ds(hh * LANES, LANES)
            oldk_v[j, sl] = av * oldk_v[j, sl] + ck_v[j, sl]
            oldv_v[j, sl] = av * oldv_v[j, sl] + cv_v[j, sl]
        return carry

    lax.fori_loop(0, RPW, row, 0)

    cp3 = pltpu.async_copy(oldk_v, memk_ref.at[idx_v], sem1)
    cp4 = pltpu.async_copy(oldv_v, memv_ref.at[idx_v], sem2)
    cp3.wait()
    cp4.wait()


@functools.cache
def _make_sc_scatter():
    return pl.kernel(
        _sc_scatter_body,
        mesh=plsc.VectorSubcoreMesh(core_axis_name="c", subcore_axis_name="s"),
        scratch_types=[
            pltpu.VMEM((RPW,), jnp.int32),
            pltpu.VMEM((RPW, LANES), jnp.float32),
            pltpu.VMEM((RPW, D), jnp.float32),
            pltpu.VMEM((RPW, D), jnp.float32),
            pltpu.VMEM((RPW, D), jnp.float32),
            pltpu.VMEM((RPW, D), jnp.float32),
            pltpu.SemaphoreType.DMA,
            pltpu.SemaphoreType.DMA,
            pltpu.SemaphoreType.DMA,
        ],
    )


def _sc_scatter(mkf, mvf, fidx, a2, ck2, cv2):
    _make_sc_scatter()(mkf, mvf, fidx, a2, ck2, cv2)


def kernel(s, mem_k, mem_v, Wq, W_wl, b_wl, Wk, Wv, Wr, br, W1, W2, Wg, bg):
    (mk_copy, mv_copy, s_new, A3, Ck3, Cv3, fidx) = _stream(
        s, mem_k, mem_v, Wq, W_wl, b_wl, Wk, Wv, Wr, br, W1, W2, Wg, bg)

    mkf = jax.new_ref(mk_copy.reshape(B * S, D))
    mvf = jax.new_ref(mv_copy.reshape(B * S, D))
    _sc_scatter(mkf, mvf,
                fidx.reshape(B * KW),
                A3.reshape(B * KW, LANES),
                Ck3.reshape(B * KW, D),
                Cv3.reshape(B * KW, D))
    mem_k_new = jax.freeze(mkf).reshape(B, S, D)
    mem_v_new = jax.freeze(mvf).reshape(B, S, D)
    return s_new, mem_k_new, mem_v_new
